# Initial kernel scaffold; baseline (speedup 1.0000x reference)
import functools

import jax
import jax.numpy as jnp
from jax import lax
from jax.experimental import pallas as pl
from jax.experimental.pallas import tpu as pltpu
from jax.experimental.pallas import tpu_sc as plsc

B, N, E, C = 4, 10000, 320000, 128
NC, NS = 2, 16
N_PAD = 10240
ROWS_T = N_PAD // NS
CH = 80
ET = E // NS
NCH = ET // CH
DROWS = N_PAD // 16
DCH = DROWS // 128


def _sc_aggregate(xf, src2, dst2, w2):
    mesh = plsc.VectorSubcoreMesh(core_axis_name="c", subcore_axis_name="s")

    @functools.partial(
        pl.kernel,
        out_type=jax.ShapeDtypeStruct((B, N_PAD, C), jnp.float32),
        mesh=mesh,
        scratch_types=[
            pltpu.VMEM((NCH, CH), jnp.int32),
            pltpu.VMEM((NCH, CH), jnp.int32),
            pltpu.VMEM((NCH, CH), jnp.float32),
            pltpu.VMEM((CH, C), jnp.float32),
            pltpu.VMEM((CH, C), jnp.float32),
            pltpu.VMEM((DROWS, 16), jnp.float32),
            pltpu.VMEM((DCH, 128), jnp.int32),
            pltpu.VMEM_SHARED((N_PAD, C), jnp.float32),
            pltpu.VMEM_SHARED((DROWS, 16), jnp.float32),
            pltpu.SemaphoreType.DMA,
            pltpu.SemaphoreType.DMA,
        ],
    )
    def agg_kernel(x_hbm, src_hbm, dst_hbm, w_hbm, out_hbm,
                   src_v, dst_v, w_v, rows0, rows1, deg_v, didx_v,
                   agg_s, deg_s, sem0, sem1):
        c = lax.axis_index("c")
        s = lax.axis_index("s")
        zeros16 = jnp.zeros((16,), jnp.float32)
        ones16 = jnp.ones((16,), jnp.float32)
        iota16 = lax.iota(jnp.int32, 16)

        pltpu.sync_copy(src_hbm.at[pl.ds(s * NCH, NCH)], src_v)
        pltpu.sync_copy(dst_hbm.at[pl.ds(s * NCH, NCH)], dst_v)
        pltpu.sync_copy(w_hbm.at[pl.ds(s * NCH, NCH)], w_v)

        def zero_deg(r, _):
            deg_v[r, :] = zeros16
            return 0
        lax.fori_loop(0, DROWS, zero_deg, 0)

        @pl.when(s == 0)
        def _():
            pltpu.sync_copy(deg_v, deg_s)

        for j in range(DCH):
            for k in range(8):
                didx_v[j, pl.ds(k * 16, 16)] = iota16 + (j * 128 + k * 16)

        plsc.subcore_barrier()

        def deg_chunk(j, _):
            for k in range(CH // 16):
                d = dst_v[j, pl.ds(k * 16, 16)]
                rr = lax.shift_right_logical(d, 4)
                cc = lax.bitwise_and(d, 15)
                plsc.addupdate_scatter(deg_v, [rr, cc], ones16)
            return 0
        lax.fori_loop(0, NCH, deg_chunk, 0)

        for j in range(DCH):
            pltpu.sync_copy(deg_v.at[pl.ds(j * 128, 128)],
                            deg_s.at[didx_v.at[j]], add=True)
        plsc.subcore_barrier()

        dslab = ROWS_T // 16
        pltpu.sync_copy(deg_s.at[pl.ds(dslab * s, dslab)], deg_v.at[pl.ds(0, dslab)])
        def recip(r, _):
            deg_v[r, :] = 1.0 / jnp.maximum(deg_v[r, :], 1.0)
            return 0
        lax.fori_loop(0, dslab, recip, 0)

        def gather(jj, buf, sem):
            return pltpu.async_copy(x_hbm.at[src_v.at[jj]], buf, sem)

        def gwait(buf, sem):
            pltpu.make_async_copy(x_hbm.at[src_v.at[0]], buf, sem).wait()

        def scale_scatter(jj, buf):
            def per_edge(e, _):
                w = w_v[jj, e]
                for k in range(C // 16):
                    buf[e, pl.ds(k * 16, 16)] = buf[e, pl.ds(k * 16, 16)] * w
                return 0
            lax.fori_loop(0, CH, per_edge, 0)
            pltpu.sync_copy(buf, agg_s.at[dst_v.at[jj]], add=True)

        for p in range(2):
            b = c * 2 + p
            delta = jnp.where(jnp.int32(p) == 0, c * 2 * N, N).astype(jnp.int32)
            def shift_chunk(j, _):
                for k in range(CH // 16):
                    src_v[j, pl.ds(k * 16, 16)] = src_v[j, pl.ds(k * 16, 16)] + delta
                return 0
            lax.fori_loop(0, NCH, shift_chunk, 0)

            def zero_rows(r, _):
                for k in range(C // 16):
                    rows0[r, pl.ds(k * 16, 16)] = zeros16
                return 0
            lax.fori_loop(0, CH, zero_rows, 0)
            for k in range(ROWS_T // CH):
                pltpu.sync_copy(rows0, agg_s.at[pl.ds(ROWS_T * s + CH * k, CH)])
            plsc.subcore_barrier()

            gather(0, rows0, sem0).start()
            def chunk_pair(jj, _):
                gather(2 * jj + 1, rows1, sem1).start()
                gwait(rows0, sem0)
                scale_scatter(2 * jj, rows0)
                @pl.when(jj < NCH // 2 - 1)
                def _():
                    gather(2 * jj + 2, rows0, sem0).start()
                gwait(rows1, sem1)
                scale_scatter(2 * jj + 1, rows1)
                return 0
            lax.fori_loop(0, NCH // 2, chunk_pair, 0)
            plsc.subcore_barrier()

            for k in range(ROWS_T // CH):
                pltpu.sync_copy(agg_s.at[pl.ds(ROWS_T * s + CH * k, CH)], rows0)
                def mean_row(e, _):
                    r = k * CH + e
                    rec = deg_v[lax.shift_right_logical(r, 4), lax.bitwise_and(r, 15)]
                    for q in range(C // 16):
                        rows0[e, pl.ds(q * 16, 16)] = rows0[e, pl.ds(q * 16, 16)] * rec
                    return 0
                lax.fori_loop(0, CH, mean_row, 0)
                pltpu.sync_copy(rows0, out_hbm.at[b, pl.ds(ROWS_T * s + CH * k, CH)])

    return agg_kernel(xf, src2, dst2, w2)


def _tc_dense(agg, x, W_rel, b_rel, W_root):
    BLK = 1000
    grid = (B, N // BLK)

    def body(agg_ref, x_ref, wrel_ref, wroot_ref, brel_ref, out_ref):
        a = agg_ref[0]
        xb = x_ref[0]
        dn = (((1,), (1,)), ((), ()))
        out_ref[0] = (
            lax.dot_general(a, wrel_ref[...], dn,
                            preferred_element_type=jnp.float32)
            + lax.dot_general(xb, wroot_ref[...], dn,
                              preferred_element_type=jnp.float32)
            + brel_ref[...]
        )

    return pl.pallas_call(
        body,
        grid=grid,
        in_specs=[
            pl.BlockSpec((1, BLK, C), lambda b, i: (b, i, 0)),
            pl.BlockSpec((1, BLK, C), lambda b, i: (b, i, 0)),
            pl.BlockSpec((C, C), lambda b, i: (0, 0)),
            pl.BlockSpec((C, C), lambda b, i: (0, 0)),
            pl.BlockSpec((1, C), lambda b, i: (0, 0)),
        ],
        out_specs=pl.BlockSpec((1, BLK, C), lambda b, i: (b, i, 0)),
        out_shape=jax.ShapeDtypeStruct((B, N, C), jnp.float32),
    )(agg, x, W_rel, W_root, b_rel.reshape(1, C))


def kernel(x, index, weight, W_rel, b_rel, W_root):
    xf = x.reshape(B * N, C)
    src2 = index[0].reshape(E // CH, CH)
    dst2 = index[1].reshape(E // CH, CH)
    w2 = weight.reshape(E // CH, CH)
    agg = _sc_aggregate(xf, src2, dst2, w2)
    return _tc_dense(agg, x, W_rel, b_rel, W_root)


# speedup vs baseline: 24.6961x; 24.6961x over previous
import functools

import jax
import jax.numpy as jnp
from jax import lax
from jax.experimental import pallas as pl
from jax.experimental.pallas import tpu as pltpu
from jax.experimental.pallas import tpu_sc as plsc

B, N, E, C = 4, 10000, 320000, 128
NC, NS = 2, 16
N_PAD = 10240
ROWS_T = N_PAD // NS
CH = 128
NCH = 158
E_PAD = NS * NCH * CH
CC = C // 2
DROWS = N_PAD // 16
DCH = DROWS // 128
G = CH // 16


def _sc_aggregate(xg, meta):
    mesh = plsc.VectorSubcoreMesh(core_axis_name="c", subcore_axis_name="s")

    @functools.partial(
        pl.kernel,
        out_type=jax.ShapeDtypeStruct((B, 2, N_PAD, CC), jnp.float32),
        mesh=mesh,
        compiler_params=pltpu.CompilerParams(needs_layout_passes=False,
                                             use_tc_tiling_on_sc=False),
        scratch_types=[
            pltpu.VMEM((3, CH), jnp.int32),
            pltpu.VMEM((3, CH), jnp.int32),
            pltpu.VMEM((2, CH), jnp.int32),
            pltpu.VMEM((CH, CC), jnp.float32),
            pltpu.VMEM((CH, CC), jnp.float32),
            pltpu.VMEM((DROWS, 16), jnp.float32),
            pltpu.VMEM((DCH, 128), jnp.int32),
            pltpu.VMEM_SHARED((N_PAD, CC), jnp.float32),
            pltpu.VMEM_SHARED((DROWS, 16), jnp.float32),
            pltpu.SemaphoreType.DMA,
            pltpu.SemaphoreType.DMA,
            pltpu.SemaphoreType.DMA,
            pltpu.SemaphoreType.DMA,
        ],
    )
    def agg_kernel(x_hbm, meta_hbm, out_hbm,
                   meta0, meta1, srcsh_v, rows0, rows1, deg_v, didx_v,
                   agg_s, deg_s, sem0, sem1, semi0, semi1):
        c = lax.axis_index("c")
        s = lax.axis_index("s")
        zeros16 = jnp.zeros((16,), jnp.float32)
        ones16 = jnp.ones((16,), jnp.float32)
        iota16 = lax.iota(jnp.int32, 16)
        rows = (rows0, rows1)
        metas = (meta0, meta1)
        gsem = (sem0, sem1)
        isem = (semi0, semi1)

        def zero_deg(r, _):
            deg_v[r, :] = zeros16
            return 0
        lax.fori_loop(0, DROWS, zero_deg, 0)

        @pl.when(s == 0)
        def _():
            pltpu.sync_copy(deg_v, deg_s)

        for j in range(DCH):
            for k in range(8):
                didx_v[j, pl.ds(k * 16, 16)] = iota16 + (j * 128 + k * 16)

        def zero_rows(r, _):
            for k in range(CC // 16):
                rows0[r, pl.ds(k * 16, 16)] = zeros16
            return 0

        def idx_issue(j, slot):
            pltpu.async_copy(meta_hbm.at[s, j], metas[slot], isem[slot])

        def idx_wait(slot):
            pltpu.make_async_copy(meta_hbm.at[s, 0], metas[slot],
                                  isem[slot]).wait()

        def gather_issue(h, slot):
            pltpu.async_copy(x_hbm.at[h].at[srcsh_v.at[slot]], rows[slot],
                             gsem[slot])

        def gather_wait(h, slot):
            pltpu.make_async_copy(x_hbm.at[h].at[srcsh_v.at[0]], rows[slot],
                                  gsem[slot]).wait()

        def shift_src(slot, delta):
            for k in range(G):
                srcsh_v[slot, pl.ds(k * 16, 16)] = (
                    metas[slot][0, pl.ds(k * 16, 16)] + delta)

        def deg_scatter(slot):
            for k in range(G):
                d = metas[slot][1, pl.ds(k * 16, 16)]
                rr = lax.shift_right_logical(d, 4)
                cc = lax.bitwise_and(d, 15)
                plsc.addupdate_scatter(deg_v, [rr, cc], ones16)

        def scale(slot):
            buf = rows[slot]
            def per_group(k, _):
                w16 = plsc.bitcast(metas[slot][2, pl.ds(k * 16, 16)],
                                   jnp.float32)
                for lane in range(16):
                    w = w16[lane]
                    e = k * 16 + lane
                    for q in range(CC // 16):
                        buf[e, pl.ds(q * 16, 16)] = (
                            buf[e, pl.ds(q * 16, 16)] * w)
                return 0
            lax.fori_loop(0, G, per_group, 0)

        def scatter_add(slot):
            pltpu.sync_copy(rows[slot], agg_s.at[metas[slot].at[1]],
                            add=True)

        for p in range(2):
            b = c * 2 + p
            delta = (b * N).astype(jnp.int32)
            for h in range(2):
                first = (p == 0 and h == 0)

                lax.fori_loop(0, CH, zero_rows, 0)
                for k in range(ROWS_T // CH):
                    pltpu.sync_copy(rows0,
                                    agg_s.at[pl.ds(ROWS_T * s + CH * k, CH)])
                plsc.subcore_barrier()

                idx_issue(0, 0)
                idx_issue(1, 1)

                def chunk_pair(jj, _):
                    idx_wait(0)
                    shift_src(0, delta)
                    gather_issue(h, 0)
                    idx_wait(1)
                    shift_src(1, delta)
                    gather_issue(h, 1)
                    gather_wait(h, 0)
                    if first:
                        deg_scatter(0)
                    scale(0)
                    scatter_add(0)
                    @pl.when(jj < NCH // 2 - 1)
                    def _():
                        idx_issue(2 * jj + 2, 0)
                    gather_wait(h, 1)
                    if first:
                        deg_scatter(1)
                    scale(1)
                    scatter_add(1)
                    @pl.when(jj < NCH // 2 - 1)
                    def _():
                        idx_issue(2 * jj + 3, 1)
                    return 0
                lax.fori_loop(0, NCH // 2, chunk_pair, 0)
                plsc.subcore_barrier()

                if first:
                    for j in range(DCH):
                        pltpu.sync_copy(deg_v.at[pl.ds(j * 128, 128)],
                                        deg_s.at[didx_v.at[j]], add=True)
                    plsc.subcore_barrier()
                    dslab = ROWS_T // 16
                    pltpu.sync_copy(deg_s.at[pl.ds(dslab * s, dslab)],
                                    deg_v.at[pl.ds(0, dslab)])
                    def recip(r, _):
                        deg_v[r, :] = 1.0 / jnp.maximum(deg_v[r, :], 1.0)
                        return 0
                    lax.fori_loop(0, dslab, recip, 0)

                def wchunk(k, _):
                    pltpu.sync_copy(agg_s.at[pl.ds(ROWS_T * s + CH * k, CH)],
                                    rows0)
                    def mean_group(gg, _):
                        rec16 = deg_v[G * k + gg, :]
                        for lane in range(16):
                            rec = rec16[lane]
                            e = gg * 16 + lane
                            for q in range(CC // 16):
                                rows0[e, pl.ds(q * 16, 16)] = (
                                    rows0[e, pl.ds(q * 16, 16)] * rec)
                        return 0
                    lax.fori_loop(0, G, mean_group, 0)
                    pltpu.sync_copy(
                        rows0,
                        out_hbm.at[b, h, pl.ds(ROWS_T * s + CH * k, CH)])
                    return 0
                lax.fori_loop(0, ROWS_T // CH, wchunk, 0)

    return agg_kernel(xg, meta)


def _tc_dense(agg, x, W_rel, b_rel, W_root):
    BLK = 1000
    grid = (B, N // BLK)

    def body(a0_ref, a1_ref, x_ref, wr0_ref, wr1_ref, wroot_ref, brel_ref,
             out_ref):
        dn = (((1,), (1,)), ((), ()))
        out_ref[0] = (
            lax.dot_general(a0_ref[0, 0], wr0_ref[...], dn,
                            preferred_element_type=jnp.float32)
            + lax.dot_general(a1_ref[0, 0], wr1_ref[...], dn,
                              preferred_element_type=jnp.float32)
            + lax.dot_general(x_ref[0], wroot_ref[...], dn,
                              preferred_element_type=jnp.float32)
            + brel_ref[...]
        )

    return pl.pallas_call(
        body,
        grid=grid,
        in_specs=[
            pl.BlockSpec((1, 1, BLK, CC), lambda b, i: (b, 0, i, 0)),
            pl.BlockSpec((1, 1, BLK, CC), lambda b, i: (b, 1, i, 0)),
            pl.BlockSpec((1, BLK, C), lambda b, i: (b, i, 0)),
            pl.BlockSpec((C, CC), lambda b, i: (0, 0)),
            pl.BlockSpec((C, CC), lambda b, i: (0, 0)),
            pl.BlockSpec((C, C), lambda b, i: (0, 0)),
            pl.BlockSpec((1, C), lambda b, i: (0, 0)),
        ],
        out_specs=pl.BlockSpec((1, BLK, C), lambda b, i: (b, i, 0)),
        out_shape=jax.ShapeDtypeStruct((B, N, C), jnp.float32),
    )(agg, agg, x, W_rel[:, :CC], W_rel[:, CC:], W_root, b_rel.reshape(1, C))


def kernel(x, index, weight, W_rel, b_rel, W_root):
    xf = x.reshape(B * N, C)
    xg = jnp.stack([xf[:, :CC], xf[:, CC:]], axis=0)
    pad = E_PAD - E
    srcp = jnp.concatenate([index[0], jnp.zeros((pad,), jnp.int32)])
    dstp = jnp.concatenate([index[1], jnp.full((pad,), N, jnp.int32)])
    wp = jnp.concatenate([weight, jnp.zeros((pad,), jnp.float32)])
    src3 = srcp.reshape(NS, NCH, CH)
    dst3 = dstp.reshape(NS, NCH, CH)
    wbits = lax.bitcast_convert_type(wp, jnp.int32).reshape(NS, NCH, CH)
    meta = jnp.stack([src3, dst3, wbits], axis=2)
    agg = _sc_aggregate(xg, meta)
    return _tc_dense(agg, x, W_rel, b_rel, W_root)
